# Initial kernel scaffold; baseline (speedup 1.0000x reference)
import jax
import jax.numpy as jnp
from jax.experimental import pallas as pl


def kernel(x, edge_index, edge_attr, W1, b1, W2, b2):
    raise NotImplementedError("write your pallas kernel here")


# speedup vs baseline: 7.1380x; 7.1380x over previous
import functools
import math

import numpy as np
import jax
import jax.numpy as jnp
from jax import lax
from jax.experimental import pallas as pl
from jax.experimental.pallas import tpu as pltpu
from jax.experimental.pallas import tpu_sc as plsc

_LS = (4, 6)
_NCOMP = 22
_NCOLS = 32
_COUNT_COL = 22
_NC, _NS = 2, 16
_CHUNK = 2048


def _w3j(j1, j2, j3, m1, m2, m3):
    if m1 + m2 + m3 != 0:
        return 0.0
    f = math.factorial
    delta = math.sqrt(f(j1 + j2 - j3) * f(j1 - j2 + j3) * f(-j1 + j2 + j3)
                      / f(j1 + j2 + j3 + 1))
    pref = delta * math.sqrt(f(j1 + m1) * f(j1 - m1) * f(j2 + m2) * f(j2 - m2)
                             * f(j3 + m3) * f(j3 - m3))
    tmin = max(0, j2 - j3 - m1, j1 - j3 + m2)
    tmax = min(j1 + j2 - j3, j1 - m1, j2 + m2)
    s = 0.0
    for t in range(tmin, tmax + 1):
        s += ((-1.0) ** t) / (f(t) * f(j3 - j2 + t + m1) * f(j3 - j1 + t - m2)
                              * f(j1 + j2 - j3 - t) * f(j1 - t - m1) * f(j2 - t + m2))
    return ((-1.0) ** (j1 - j2 - m3)) * pref * s


def _wigner_table(l):
    i1, i2, i3, c = [], [], [], []
    for m1 in range(-l, l + 1):
        for m2 in range(-l, l + 1):
            m3 = -m1 - m2
            if abs(m3) > l:
                continue
            v = _w3j(l, l, l, m1, m2, m3)
            if v != 0.0:
                i1.append(m1 + l)
                i2.append(m2 + l)
                i3.append(m3 + l)
                c.append(np.float32(v))
    return i1, i2, i3, c


_TABLES = [_wigner_table(l) for l in _LS]


def _dfact(n):
    r = 1.0
    while n > 1:
        r *= n
        n -= 2
    return r


def _edge_comps(xc, yc, zc):
    r2 = xc * xc + yc * yc + zc * zc
    r = jnp.sqrt(r2)
    ct = zc / r
    st = jnp.sqrt(jnp.clip(1.0 - ct * ct, 0.0, 1.0))
    rho = jnp.sqrt(xc * xc + yc * yc)
    safe = rho > 0.0
    cp = jnp.where(safe, xc / rho, 1.0)
    sp = jnp.where(safe, yc / rho, 0.0)

    lmax = max(_LS)
    er = [None] * (lmax + 1)
    ei = [None] * (lmax + 1)
    er[1], ei[1] = cp, sp
    for m in range(2, lmax + 1):
        er[m] = er[m - 1] * cp - ei[m - 1] * sp
        ei[m] = er[m - 1] * sp + ei[m - 1] * cp

    stp = [None] * (lmax + 1)
    if lmax >= 1:
        stp[1] = st
    for m in range(2, lmax + 1):
        stp[m] = stp[m - 1] * st

    P = {}
    for m in range(0, lmax + 1):
        sgn_df = ((-1.0) ** m) * _dfact(2 * m - 1)
        if m == 0:
            pmm = jnp.ones_like(ct)
        else:
            pmm = sgn_df * stp[m]
        prev, cur = pmm, None
        P[(m, m)] = pmm
        if m < lmax:
            cur = ct * float(2 * m + 1) * pmm
            P[(m + 1, m)] = cur
        for ll in range(m + 2, lmax + 1):
            nxt = (float(2 * ll - 1) * ct * cur - float(ll + m - 1) * prev) / float(ll - m)
            prev, cur = cur, nxt
            P[(ll, m)] = nxt

    comps = []
    for l in _LS:
        for m in range(0, l + 1):
            norm = math.sqrt((2 * l + 1) / (4.0 * math.pi)
                             * math.factorial(l - m) / math.factorial(l + m))
            base = norm * P[(l, m)]
            if m == 0:
                comps.append(base)
            else:
                comps.append(base * er[m])
                comps.append(base * ei[m])
    comps.append(jnp.ones_like(xc))
    return comps


def _node_outputs(q, getw1, getb1, getw2, getb2):
    qls, wls = [], []
    off = 0
    for l, (i1, i2, i3, cs) in zip(_LS, _TABLES):
        qr = [None] * (2 * l + 1)
        qi = [None] * (2 * l + 1)
        zero = jnp.zeros_like(q[off])
        qr[l] = q[off]
        qi[l] = zero
        idx = off + 1
        norm2 = qr[l] * qr[l]
        for m in range(1, l + 1):
            rr, ii = q[idx], q[idx + 1]
            idx += 2
            qr[l + m] = rr
            qi[l + m] = ii
            sgn = (-1.0) ** m
            qr[l - m] = sgn * rr
            qi[l - m] = (-sgn) * ii
            norm2 = norm2 + 2.0 * (rr * rr + ii * ii)
        off = idx
        wsum = zero
        for a, b, c, coef in zip(i1, i2, i3, cs):
            ar, ai = qr[a], qi[a]
            br, bi = qr[b], qi[b]
            cr, ci = qr[c], qi[c]
            tr = ar * br - ai * bi
            ti = ar * bi + ai * br
            wsum = wsum + float(coef) * (tr * cr - ti * ci)
        ql = jnp.sqrt((4.0 * math.pi / (2 * l + 1)) * norm2)
        p = norm2 * jnp.sqrt(norm2)
        wl = jnp.nan_to_num(wsum / p)
        qls.append(ql)
        wls.append(wl)

    emb = qls + wls
    h = []
    for j in range(64):
        pre = getb1(j)
        for k in range(4):
            pre = pre + emb[k] * getw1(k, j)
        h.append(pre * (1.0 / (1.0 + jnp.exp(-pre))))
    outs = []
    for k in range(4):
        o = getb2(k)
        for j in range(64):
            o = o + h[j] * getw2(j, k)
        outs.append(o)
    return outs, emb


def _feat_body(ea_ref, out_ref):
    xc = ea_ref[0, 0]
    yc = ea_ref[0, 1]
    zc = ea_ref[0, 2]
    comps = _edge_comps(xc, yc, zc)
    for j, cmp in enumerate(comps):
        out_ref[j, 0] = cmp
    zero = jnp.zeros_like(xc)
    for j in range(len(comps), _NCOLS):
        out_ref[j, 0] = zero


def _finish_body(a_ref, w1_ref, b1_ref, w2_ref, b2_ref, out_ref, emb_ref):
    count = a_ref[_COUNT_COL]
    inv = 1.0 / jnp.maximum(count, 1.0)
    q = [a_ref[j] * inv for j in range(_NCOMP)]
    outs, emb = _node_outputs(
        q,
        lambda k, j: w1_ref[k, j],
        lambda j: b1_ref[j],
        lambda j, k: w2_ref[j, k],
        lambda k: b2_ref[k],
    )
    for k in range(4):
        out_ref[k] = outs[k]
        emb_ref[k] = emb[k]


def _make_scatter(n_pad, ep, n_chunks):
    mesh = plsc.VectorSubcoreMesh(core_axis_name="c", subcore_axis_name="s",
                                  num_cores=_NC, num_subcores=_NS)

    @functools.partial(
        pl.kernel,
        out_type=jax.ShapeDtypeStruct((_NC * _NS * n_pad,), jnp.float32),
        mesh=mesh,
        scratch_types=[
            pltpu.VMEM((_CHUNK,), jnp.float32),
            pltpu.VMEM((_CHUNK,), jnp.int32),
            pltpu.VMEM((n_pad,), jnp.float32),
        ],
        compiler_params=pltpu.CompilerParams(needs_layout_passes=False),
    )
    def scatter(ft_hbm, dst_hbm, zeros_hbm, out_hbm, vals_v, idx_v, acc):
        c = lax.axis_index("c")
        s = lax.axis_index("s")
        w = c * _NS + s
        pltpu.sync_copy(zeros_hbm, acc)

        def chunk_body(k, carry):
            pltpu.sync_copy(ft_hbm.at[pl.ds(w * ep + k * _CHUNK, _CHUNK)],
                            vals_v)
            pltpu.sync_copy(dst_hbm.at[pl.ds(k * _CHUNK, _CHUNK)], idx_v)
            for i in range(_CHUNK // 16):
                idx = idx_v[pl.ds(i * 16, 16)]
                val = vals_v[pl.ds(i * 16, 16)]
                plsc.addupdate_scatter(acc, [idx], val)
            return carry

        lax.fori_loop(0, n_chunks, chunk_body, 0)
        pltpu.sync_copy(acc, out_hbm.at[pl.ds(w * n_pad, n_pad)])

    return scatter


def kernel(x, edge_index, edge_attr, W1, b1, W2, b2):
    n = x.shape[0]
    e = edge_attr.shape[0]
    nb = -(-e // 1024)
    cpw = -(-nb // (_NC * _NS))
    nbp = _NC * _NS * cpw
    ep = nbp * 1024
    n_pad = -(-n // 1024) * 1024

    dst = edge_index[1].astype(jnp.int32)
    pad_e = ep - e
    ea_p = jnp.concatenate(
        [edge_attr.astype(jnp.float32),
         jnp.broadcast_to(jnp.array([1.0, 0.0, 0.0], jnp.float32), (pad_e, 3))])
    dst_p = jnp.concatenate([dst, jnp.full((pad_e,), n, jnp.int32)])

    ea_t = ea_p.T.reshape(3, nbp, 8, 128).transpose(1, 0, 2, 3)
    feat = pl.pallas_call(
        _feat_body,
        grid=(nbp,),
        in_specs=[pl.BlockSpec((1, 3, 8, 128), lambda i: (i, 0, 0, 0))],
        out_specs=pl.BlockSpec((_NCOLS, 1, 8, 128), lambda i: (0, i, 0, 0)),
        out_shape=jax.ShapeDtypeStruct((_NCOLS, nbp, 8, 128), jnp.float32),
    )(ea_t)
    ft_flat = feat.reshape(_NCOLS * ep)

    n_chunks = ep // _CHUNK
    zeros = jnp.zeros((n_pad,), jnp.float32)
    colsums = _make_scatter(n_pad, ep, n_chunks)(ft_flat, dst_p, zeros)
    a_t = colsums.reshape(_NCOLS, n_pad // 128, 128)

    out_t, emb_t = pl.pallas_call(
        _finish_body,
        grid=(n_pad // 1024,),
        in_specs=[
            pl.BlockSpec((_NCOLS, 8, 128), lambda g: (0, g, 0)),
            pl.BlockSpec(memory_space=pltpu.SMEM),
            pl.BlockSpec(memory_space=pltpu.SMEM),
            pl.BlockSpec(memory_space=pltpu.SMEM),
            pl.BlockSpec(memory_space=pltpu.SMEM),
        ],
        out_specs=[
            pl.BlockSpec((4, 8, 128), lambda g: (0, g, 0)),
            pl.BlockSpec((4, 8, 128), lambda g: (0, g, 0)),
        ],
        out_shape=[
            jax.ShapeDtypeStruct((4, n_pad // 128, 128), jnp.float32),
            jax.ShapeDtypeStruct((4, n_pad // 128, 128), jnp.float32),
        ],
    )(a_t, W1, b1, W2, b2)

    out = out_t.reshape(4, n_pad)[:, :n].T
    emb = emb_t.reshape(4, n_pad)[:, :n].T
    return out, emb


# speedup vs baseline: 9.1129x; 1.2767x over previous
import functools
import math

import numpy as np
import jax
import jax.numpy as jnp
from jax import lax
from jax.experimental import pallas as pl
from jax.experimental.pallas import tpu as pltpu
from jax.experimental.pallas import tpu_sc as plsc

_LS = (4, 6)
_NCOMP = 22
_NCOLS = 32
_COUNT_COL = 22
_NC, _NS = 2, 16
_CHUNK = 8192


def _w3j(j1, j2, j3, m1, m2, m3):
    if m1 + m2 + m3 != 0:
        return 0.0
    f = math.factorial
    delta = math.sqrt(f(j1 + j2 - j3) * f(j1 - j2 + j3) * f(-j1 + j2 + j3)
                      / f(j1 + j2 + j3 + 1))
    pref = delta * math.sqrt(f(j1 + m1) * f(j1 - m1) * f(j2 + m2) * f(j2 - m2)
                             * f(j3 + m3) * f(j3 - m3))
    tmin = max(0, j2 - j3 - m1, j1 - j3 + m2)
    tmax = min(j1 + j2 - j3, j1 - m1, j2 + m2)
    s = 0.0
    for t in range(tmin, tmax + 1):
        s += ((-1.0) ** t) / (f(t) * f(j3 - j2 + t + m1) * f(j3 - j1 + t - m2)
                              * f(j1 + j2 - j3 - t) * f(j1 - t - m1) * f(j2 - t + m2))
    return ((-1.0) ** (j1 - j2 - m3)) * pref * s


def _wigner_table(l):
    i1, i2, i3, c = [], [], [], []
    for m1 in range(-l, l + 1):
        for m2 in range(-l, l + 1):
            m3 = -m1 - m2
            if abs(m3) > l:
                continue
            v = _w3j(l, l, l, m1, m2, m3)
            if v != 0.0:
                i1.append(m1 + l)
                i2.append(m2 + l)
                i3.append(m3 + l)
                c.append(np.float32(v))
    return i1, i2, i3, c


_TABLES = [_wigner_table(l) for l in _LS]


def _dfact(n):
    r = 1.0
    while n > 1:
        r *= n
        n -= 2
    return r


def _edge_comps(xc, yc, zc):
    r2 = xc * xc + yc * yc + zc * zc
    r = jnp.sqrt(r2)
    ct = zc / r
    st = jnp.sqrt(jnp.clip(1.0 - ct * ct, 0.0, 1.0))
    rho = jnp.sqrt(xc * xc + yc * yc)
    safe = rho > 0.0
    cp = jnp.where(safe, xc / rho, 1.0)
    sp = jnp.where(safe, yc / rho, 0.0)

    lmax = max(_LS)
    er = [None] * (lmax + 1)
    ei = [None] * (lmax + 1)
    er[1], ei[1] = cp, sp
    for m in range(2, lmax + 1):
        er[m] = er[m - 1] * cp - ei[m - 1] * sp
        ei[m] = er[m - 1] * sp + ei[m - 1] * cp

    stp = [None] * (lmax + 1)
    if lmax >= 1:
        stp[1] = st
    for m in range(2, lmax + 1):
        stp[m] = stp[m - 1] * st

    P = {}
    for m in range(0, lmax + 1):
        sgn_df = ((-1.0) ** m) * _dfact(2 * m - 1)
        if m == 0:
            pmm = jnp.ones_like(ct)
        else:
            pmm = sgn_df * stp[m]
        prev, cur = pmm, None
        P[(m, m)] = pmm
        if m < lmax:
            cur = ct * float(2 * m + 1) * pmm
            P[(m + 1, m)] = cur
        for ll in range(m + 2, lmax + 1):
            nxt = (float(2 * ll - 1) * ct * cur - float(ll + m - 1) * prev) / float(ll - m)
            prev, cur = cur, nxt
            P[(ll, m)] = nxt

    comps = []
    for l in _LS:
        for m in range(0, l + 1):
            norm = math.sqrt((2 * l + 1) / (4.0 * math.pi)
                             * math.factorial(l - m) / math.factorial(l + m))
            base = norm * P[(l, m)]
            if m == 0:
                comps.append(base)
            else:
                comps.append(base * er[m])
                comps.append(base * ei[m])
    comps.append(jnp.ones_like(xc))
    return comps


def _node_outputs(q, getw1, getb1, getw2, getb2):
    qls, wls = [], []
    off = 0
    for l, (i1, i2, i3, cs) in zip(_LS, _TABLES):
        qr = [None] * (2 * l + 1)
        qi = [None] * (2 * l + 1)
        zero = jnp.zeros_like(q[off])
        qr[l] = q[off]
        qi[l] = zero
        idx = off + 1
        norm2 = qr[l] * qr[l]
        for m in range(1, l + 1):
            rr, ii = q[idx], q[idx + 1]
            idx += 2
            qr[l + m] = rr
            qi[l + m] = ii
            sgn = (-1.0) ** m
            qr[l - m] = sgn * rr
            qi[l - m] = (-sgn) * ii
            norm2 = norm2 + 2.0 * (rr * rr + ii * ii)
        off = idx
        wsum = zero
        for a, b, c, coef in zip(i1, i2, i3, cs):
            ar, ai = qr[a], qi[a]
            br, bi = qr[b], qi[b]
            cr, ci = qr[c], qi[c]
            tr = ar * br - ai * bi
            ti = ar * bi + ai * br
            wsum = wsum + float(coef) * (tr * cr - ti * ci)
        ql = jnp.sqrt((4.0 * math.pi / (2 * l + 1)) * norm2)
        p = norm2 * jnp.sqrt(norm2)
        wl = jnp.nan_to_num(wsum / p)
        qls.append(ql)
        wls.append(wl)

    emb = qls + wls
    h = []
    for j in range(64):
        pre = getb1(j)
        for k in range(4):
            pre = pre + emb[k] * getw1(k, j)
        h.append(pre * (1.0 / (1.0 + jnp.exp(-pre))))
    outs = []
    for k in range(4):
        o = getb2(k)
        for j in range(64):
            o = o + h[j] * getw2(j, k)
        outs.append(o)
    return outs, emb


def _feat_body(ea_ref, out_ref):
    xc = ea_ref[0, 0]
    yc = ea_ref[0, 1]
    zc = ea_ref[0, 2]
    comps = _edge_comps(xc, yc, zc)
    for j, cmp in enumerate(comps):
        out_ref[j, 0] = cmp
    zero = jnp.zeros_like(xc)
    for j in range(len(comps), _NCOLS):
        out_ref[j, 0] = zero


def _finish_body(a_ref, w1_ref, b1_ref, w2_ref, b2_ref, out_ref, emb_ref):
    count = a_ref[_COUNT_COL]
    inv = 1.0 / jnp.maximum(count, 1.0)
    q = [a_ref[j] * inv for j in range(_NCOMP)]
    outs, emb = _node_outputs(
        q,
        lambda k, j: w1_ref[k, j],
        lambda j: b1_ref[j],
        lambda j, k: w2_ref[j, k],
        lambda k: b2_ref[k],
    )
    for k in range(4):
        out_ref[k] = outs[k]
        emb_ref[k] = emb[k]


def _make_scatter(n_pad, ep, n_chunks):
    mesh = plsc.VectorSubcoreMesh(core_axis_name="c", subcore_axis_name="s",
                                  num_cores=_NC, num_subcores=_NS)

    @functools.partial(
        pl.kernel,
        out_type=jax.ShapeDtypeStruct((_NC * _NS * n_pad,), jnp.float32),
        mesh=mesh,
        scratch_types=[
            pltpu.VMEM((_CHUNK,), jnp.float32),
            pltpu.VMEM((_CHUNK,), jnp.int32),
            pltpu.VMEM((n_pad,), jnp.float32),
        ],
        compiler_params=pltpu.CompilerParams(needs_layout_passes=False),
    )
    def scatter(ft_hbm, dst_hbm, zeros_hbm, out_hbm, vals_v, idx_v, acc):
        c = lax.axis_index("c")
        s = lax.axis_index("s")
        w = c * _NS + s
        pltpu.sync_copy(zeros_hbm, acc)

        def chunk_body(k, carry):
            pltpu.sync_copy(ft_hbm.at[pl.ds(w * ep + k * _CHUNK, _CHUNK)],
                            vals_v)
            pltpu.sync_copy(dst_hbm.at[pl.ds(k * _CHUNK, _CHUNK)], idx_v)
            for i in range(_CHUNK // 16):
                idx = idx_v[pl.ds(i * 16, 16)]
                val = vals_v[pl.ds(i * 16, 16)]
                plsc.addupdate_scatter(acc, [idx], val)
            return carry

        lax.fori_loop(0, n_chunks, chunk_body, 0)
        pltpu.sync_copy(acc, out_hbm.at[pl.ds(w * n_pad, n_pad)])

    return scatter


def kernel(x, edge_index, edge_attr, W1, b1, W2, b2):
    n = x.shape[0]
    e = edge_attr.shape[0]
    nb = -(-e // 1024)
    cpw = -(-nb // (_NC * _NS))
    nbp = _NC * _NS * cpw
    ep = nbp * 1024
    n_pad = -(-n // 1024) * 1024

    dst = edge_index[1].astype(jnp.int32)
    pad_e = ep - e
    ea_p = jnp.concatenate(
        [edge_attr.astype(jnp.float32),
         jnp.broadcast_to(jnp.array([1.0, 0.0, 0.0], jnp.float32), (pad_e, 3))])
    dst_p = jnp.concatenate([dst, jnp.full((pad_e,), n, jnp.int32)])

    ea_t = ea_p.T.reshape(3, nbp, 8, 128).transpose(1, 0, 2, 3)
    feat = pl.pallas_call(
        _feat_body,
        grid=(nbp,),
        in_specs=[pl.BlockSpec((1, 3, 8, 128), lambda i: (i, 0, 0, 0))],
        out_specs=pl.BlockSpec((_NCOLS, 1, 8, 128), lambda i: (0, i, 0, 0)),
        out_shape=jax.ShapeDtypeStruct((_NCOLS, nbp, 8, 128), jnp.float32),
    )(ea_t)
    ft_flat = feat.reshape(_NCOLS * ep)

    n_chunks = ep // _CHUNK
    zeros = jnp.zeros((n_pad,), jnp.float32)
    colsums = _make_scatter(n_pad, ep, n_chunks)(ft_flat, dst_p, zeros)
    a_t = colsums.reshape(_NCOLS, n_pad // 128, 128)

    out_t, emb_t = pl.pallas_call(
        _finish_body,
        grid=(n_pad // 1024,),
        in_specs=[
            pl.BlockSpec((_NCOLS, 8, 128), lambda g: (0, g, 0)),
            pl.BlockSpec(memory_space=pltpu.SMEM),
            pl.BlockSpec(memory_space=pltpu.SMEM),
            pl.BlockSpec(memory_space=pltpu.SMEM),
            pl.BlockSpec(memory_space=pltpu.SMEM),
        ],
        out_specs=[
            pl.BlockSpec((4, 8, 128), lambda g: (0, g, 0)),
            pl.BlockSpec((4, 8, 128), lambda g: (0, g, 0)),
        ],
        out_shape=[
            jax.ShapeDtypeStruct((4, n_pad // 128, 128), jnp.float32),
            jax.ShapeDtypeStruct((4, n_pad // 128, 128), jnp.float32),
        ],
    )(a_t, W1, b1, W2, b2)

    out = out_t.reshape(4, n_pad)[:, :n].T
    emb = emb_t.reshape(4, n_pad)[:, :n].T
    return out, emb


# speedup vs baseline: 10.2342x; 1.1230x over previous
import functools
import math

import numpy as np
import jax
import jax.numpy as jnp
from jax import lax
from jax.experimental import pallas as pl
from jax.experimental.pallas import tpu as pltpu
from jax.experimental.pallas import tpu_sc as plsc

_LS = (4, 6)
_NCOMP = 22
_NCOLS = 32
_COUNT_COL = 22
_NC, _NS = 2, 16
_CHUNK = 4096


def _w3j(j1, j2, j3, m1, m2, m3):
    if m1 + m2 + m3 != 0:
        return 0.0
    f = math.factorial
    delta = math.sqrt(f(j1 + j2 - j3) * f(j1 - j2 + j3) * f(-j1 + j2 + j3)
                      / f(j1 + j2 + j3 + 1))
    pref = delta * math.sqrt(f(j1 + m1) * f(j1 - m1) * f(j2 + m2) * f(j2 - m2)
                             * f(j3 + m3) * f(j3 - m3))
    tmin = max(0, j2 - j3 - m1, j1 - j3 + m2)
    tmax = min(j1 + j2 - j3, j1 - m1, j2 + m2)
    s = 0.0
    for t in range(tmin, tmax + 1):
        s += ((-1.0) ** t) / (f(t) * f(j3 - j2 + t + m1) * f(j3 - j1 + t - m2)
                              * f(j1 + j2 - j3 - t) * f(j1 - t - m1) * f(j2 - t + m2))
    return ((-1.0) ** (j1 - j2 - m3)) * pref * s


def _wigner_table(l):
    i1, i2, i3, c = [], [], [], []
    for m1 in range(-l, l + 1):
        for m2 in range(-l, l + 1):
            m3 = -m1 - m2
            if abs(m3) > l:
                continue
            v = _w3j(l, l, l, m1, m2, m3)
            if v != 0.0:
                i1.append(m1 + l)
                i2.append(m2 + l)
                i3.append(m3 + l)
                c.append(np.float32(v))
    return i1, i2, i3, c


_TABLES = [_wigner_table(l) for l in _LS]


def _dfact(n):
    r = 1.0
    while n > 1:
        r *= n
        n -= 2
    return r


def _edge_comps(xc, yc, zc):
    r2 = xc * xc + yc * yc + zc * zc
    r = jnp.sqrt(r2)
    ct = zc / r
    st = jnp.sqrt(jnp.clip(1.0 - ct * ct, 0.0, 1.0))
    rho = jnp.sqrt(xc * xc + yc * yc)
    safe = rho > 0.0
    cp = jnp.where(safe, xc / rho, 1.0)
    sp = jnp.where(safe, yc / rho, 0.0)

    lmax = max(_LS)
    er = [None] * (lmax + 1)
    ei = [None] * (lmax + 1)
    er[1], ei[1] = cp, sp
    for m in range(2, lmax + 1):
        er[m] = er[m - 1] * cp - ei[m - 1] * sp
        ei[m] = er[m - 1] * sp + ei[m - 1] * cp

    stp = [None] * (lmax + 1)
    if lmax >= 1:
        stp[1] = st
    for m in range(2, lmax + 1):
        stp[m] = stp[m - 1] * st

    P = {}
    for m in range(0, lmax + 1):
        sgn_df = ((-1.0) ** m) * _dfact(2 * m - 1)
        if m == 0:
            pmm = jnp.ones_like(ct)
        else:
            pmm = sgn_df * stp[m]
        prev, cur = pmm, None
        P[(m, m)] = pmm
        if m < lmax:
            cur = ct * float(2 * m + 1) * pmm
            P[(m + 1, m)] = cur
        for ll in range(m + 2, lmax + 1):
            nxt = (float(2 * ll - 1) * ct * cur - float(ll + m - 1) * prev) / float(ll - m)
            prev, cur = cur, nxt
            P[(ll, m)] = nxt

    comps = []
    for l in _LS:
        for m in range(0, l + 1):
            norm = math.sqrt((2 * l + 1) / (4.0 * math.pi)
                             * math.factorial(l - m) / math.factorial(l + m))
            base = norm * P[(l, m)]
            if m == 0:
                comps.append(base)
            else:
                comps.append(base * er[m])
                comps.append(base * ei[m])
    comps.append(jnp.ones_like(xc))
    return comps


def _node_outputs(q, getw1, getb1, getw2, getb2):
    qls, wls = [], []
    off = 0
    for l, (i1, i2, i3, cs) in zip(_LS, _TABLES):
        qr = [None] * (2 * l + 1)
        qi = [None] * (2 * l + 1)
        zero = jnp.zeros_like(q[off])
        qr[l] = q[off]
        qi[l] = zero
        idx = off + 1
        norm2 = qr[l] * qr[l]
        for m in range(1, l + 1):
            rr, ii = q[idx], q[idx + 1]
            idx += 2
            qr[l + m] = rr
            qi[l + m] = ii
            sgn = (-1.0) ** m
            qr[l - m] = sgn * rr
            qi[l - m] = (-sgn) * ii
            norm2 = norm2 + 2.0 * (rr * rr + ii * ii)
        off = idx
        wsum = zero
        for a, b, c, coef in zip(i1, i2, i3, cs):
            ar, ai = qr[a], qi[a]
            br, bi = qr[b], qi[b]
            cr, ci = qr[c], qi[c]
            tr = ar * br - ai * bi
            ti = ar * bi + ai * br
            wsum = wsum + float(coef) * (tr * cr - ti * ci)
        ql = jnp.sqrt((4.0 * math.pi / (2 * l + 1)) * norm2)
        p = norm2 * jnp.sqrt(norm2)
        wl = jnp.nan_to_num(wsum / p)
        qls.append(ql)
        wls.append(wl)

    emb = qls + wls
    h = []
    for j in range(64):
        pre = getb1(j)
        for k in range(4):
            pre = pre + emb[k] * getw1(k, j)
        h.append(pre * (1.0 / (1.0 + jnp.exp(-pre))))
    outs = []
    for k in range(4):
        o = getb2(k)
        for j in range(64):
            o = o + h[j] * getw2(j, k)
        outs.append(o)
    return outs, emb


def _feat_body(ea_ref, out_ref):
    xc = ea_ref[0, 0]
    yc = ea_ref[0, 1]
    zc = ea_ref[0, 2]
    comps = _edge_comps(xc, yc, zc)
    for j, cmp in enumerate(comps):
        out_ref[j, 0] = cmp
    zero = jnp.zeros_like(xc)
    for j in range(len(comps), _NCOLS):
        out_ref[j, 0] = zero


def _finish_body(a_ref, w1_ref, b1_ref, w2_ref, b2_ref, out_ref, emb_ref):
    count = a_ref[_COUNT_COL]
    inv = 1.0 / jnp.maximum(count, 1.0)
    q = [a_ref[j] * inv for j in range(_NCOMP)]
    outs, emb = _node_outputs(
        q,
        lambda k, j: w1_ref[k, j],
        lambda j: b1_ref[j],
        lambda j, k: w2_ref[j, k],
        lambda k: b2_ref[k],
    )
    for k in range(4):
        out_ref[k] = outs[k]
        emb_ref[k] = emb[k]


def _make_scatter(n_pad, ep, n_chunks):
    mesh = plsc.VectorSubcoreMesh(core_axis_name="c", subcore_axis_name="s",
                                  num_cores=_NC, num_subcores=_NS)

    @functools.partial(
        pl.kernel,
        out_type=jax.ShapeDtypeStruct((_NC * _NS * n_pad,), jnp.float32),
        mesh=mesh,
        scratch_types=[
            pltpu.VMEM((2, _CHUNK), jnp.float32),
            pltpu.VMEM((2, _CHUNK), jnp.int32),
            pltpu.VMEM((n_pad,), jnp.float32),
            pltpu.SemaphoreType.DMA,
            pltpu.SemaphoreType.DMA,
            pltpu.SemaphoreType.DMA,
            pltpu.SemaphoreType.DMA,
        ],
        compiler_params=pltpu.CompilerParams(needs_layout_passes=False),
    )
    def scatter(ft_hbm, dst_hbm, zeros_hbm, out_hbm, vals_v, idx_v, acc,
                vs0, vs1, is0, is1):
        c = lax.axis_index("c")
        s = lax.axis_index("s")
        w = c * _NS + s
        vsem = (vs0, vs1)
        isem = (is0, is1)

        def start(slot, k):
            pltpu.async_copy(ft_hbm.at[pl.ds(w * ep + k * _CHUNK, _CHUNK)],
                             vals_v.at[slot], vsem[slot])
            pltpu.async_copy(dst_hbm.at[pl.ds(k * _CHUNK, _CHUNK)],
                             idx_v.at[slot], isem[slot])

        def wait(slot):
            pltpu.make_async_copy(ft_hbm.at[pl.ds(0, _CHUNK)],
                                  vals_v.at[slot], vsem[slot]).wait()
            pltpu.make_async_copy(dst_hbm.at[pl.ds(0, _CHUNK)],
                                  idx_v.at[slot], isem[slot]).wait()

        def process(slot):
            for i in range(_CHUNK // 16):
                idx = idx_v[slot, pl.ds(i * 16, 16)]
                val = vals_v[slot, pl.ds(i * 16, 16)]
                plsc.addupdate_scatter(acc, [idx], val)

        pltpu.sync_copy(zeros_hbm, acc)
        start(0, 0)

        def chunk_body(k2, carry):
            k0 = 2 * k2
            start(1, k0 + 1)
            wait(0)
            process(0)

            @pl.when(k0 + 2 < n_chunks)
            def _():
                start(0, k0 + 2)

            wait(1)
            process(1)
            return carry

        lax.fori_loop(0, n_chunks // 2, chunk_body, 0)
        pltpu.sync_copy(acc, out_hbm.at[pl.ds(w * n_pad, n_pad)])

    return scatter


def kernel(x, edge_index, edge_attr, W1, b1, W2, b2):
    n = x.shape[0]
    e = edge_attr.shape[0]
    nb = -(-e // 1024)
    cpw = -(-nb // (_NC * _NS))
    nbp = _NC * _NS * cpw
    ep = nbp * 1024
    n_pad = -(-n // 1024) * 1024

    dst = edge_index[1].astype(jnp.int32)
    pad_e = ep - e
    ea_p = jnp.concatenate(
        [edge_attr.astype(jnp.float32),
         jnp.broadcast_to(jnp.array([1.0, 0.0, 0.0], jnp.float32), (pad_e, 3))])
    dst_p = jnp.concatenate([dst, jnp.full((pad_e,), n, jnp.int32)])

    ea_t = ea_p.T.reshape(3, nbp, 8, 128).transpose(1, 0, 2, 3)
    feat = pl.pallas_call(
        _feat_body,
        grid=(nbp,),
        in_specs=[pl.BlockSpec((1, 3, 8, 128), lambda i: (i, 0, 0, 0))],
        out_specs=pl.BlockSpec((_NCOLS, 1, 8, 128), lambda i: (0, i, 0, 0)),
        out_shape=jax.ShapeDtypeStruct((_NCOLS, nbp, 8, 128), jnp.float32),
    )(ea_t)
    ft_flat = feat.reshape(_NCOLS * ep)

    n_chunks = ep // _CHUNK
    zeros = jnp.zeros((n_pad,), jnp.float32)
    colsums = _make_scatter(n_pad, ep, n_chunks)(ft_flat, dst_p, zeros)
    a_t = colsums.reshape(_NCOLS, n_pad // 128, 128)

    out_t, emb_t = pl.pallas_call(
        _finish_body,
        grid=(n_pad // 1024,),
        in_specs=[
            pl.BlockSpec((_NCOLS, 8, 128), lambda g: (0, g, 0)),
            pl.BlockSpec(memory_space=pltpu.SMEM),
            pl.BlockSpec(memory_space=pltpu.SMEM),
            pl.BlockSpec(memory_space=pltpu.SMEM),
            pl.BlockSpec(memory_space=pltpu.SMEM),
        ],
        out_specs=[
            pl.BlockSpec((4, 8, 128), lambda g: (0, g, 0)),
            pl.BlockSpec((4, 8, 128), lambda g: (0, g, 0)),
        ],
        out_shape=[
            jax.ShapeDtypeStruct((4, n_pad // 128, 128), jnp.float32),
            jax.ShapeDtypeStruct((4, n_pad // 128, 128), jnp.float32),
        ],
    )(a_t, W1, b1, W2, b2)

    out = out_t.reshape(4, n_pad)[:, :n].T
    emb = emb_t.reshape(4, n_pad)[:, :n].T
    return out, emb


# speedup vs baseline: 12.6354x; 1.2346x over previous
import functools
import math

import numpy as np
import jax
import jax.numpy as jnp
from jax import lax
from jax.experimental import pallas as pl
from jax.experimental.pallas import tpu as pltpu
from jax.experimental.pallas import tpu_sc as plsc

_LS = (4, 6)
_NCOMP = 22
_NCOLS = 32
_COUNT_COL = 22
_NC, _NS = 2, 16
_CHUNK = 4096


def _w3j(j1, j2, j3, m1, m2, m3):
    if m1 + m2 + m3 != 0:
        return 0.0
    f = math.factorial
    delta = math.sqrt(f(j1 + j2 - j3) * f(j1 - j2 + j3) * f(-j1 + j2 + j3)
                      / f(j1 + j2 + j3 + 1))
    pref = delta * math.sqrt(f(j1 + m1) * f(j1 - m1) * f(j2 + m2) * f(j2 - m2)
                             * f(j3 + m3) * f(j3 - m3))
    tmin = max(0, j2 - j3 - m1, j1 - j3 + m2)
    tmax = min(j1 + j2 - j3, j1 - m1, j2 + m2)
    s = 0.0
    for t in range(tmin, tmax + 1):
        s += ((-1.0) ** t) / (f(t) * f(j3 - j2 + t + m1) * f(j3 - j1 + t - m2)
                              * f(j1 + j2 - j3 - t) * f(j1 - t - m1) * f(j2 - t + m2))
    return ((-1.0) ** (j1 - j2 - m3)) * pref * s


def _wigner_table(l):
    i1, i2, i3, c = [], [], [], []
    for m1 in range(-l, l + 1):
        for m2 in range(-l, l + 1):
            m3 = -m1 - m2
            if abs(m3) > l:
                continue
            v = _w3j(l, l, l, m1, m2, m3)
            if v != 0.0:
                i1.append(m1 + l)
                i2.append(m2 + l)
                i3.append(m3 + l)
                c.append(np.float32(v))
    return i1, i2, i3, c


_TABLES = [_wigner_table(l) for l in _LS]


def _dfact(n):
    r = 1.0
    while n > 1:
        r *= n
        n -= 2
    return r


def _edge_comps(xc, yc, zc):
    r2 = xc * xc + yc * yc + zc * zc
    ct = zc * lax.rsqrt(r2)
    st = jnp.sqrt(jnp.clip(1.0 - ct * ct, 0.0, 1.0))
    rho2 = xc * xc + yc * yc
    safe = rho2 > 0.0
    rinv = lax.rsqrt(rho2)
    cp = jnp.where(safe, xc * rinv, 1.0)
    sp = jnp.where(safe, yc * rinv, 0.0)

    lmax = max(_LS)
    er = [None] * (lmax + 1)
    ei = [None] * (lmax + 1)
    er[1], ei[1] = cp, sp
    for m in range(2, lmax + 1):
        er[m] = er[m - 1] * cp - ei[m - 1] * sp
        ei[m] = er[m - 1] * sp + ei[m - 1] * cp

    stp = [None] * (lmax + 1)
    if lmax >= 1:
        stp[1] = st
    for m in range(2, lmax + 1):
        stp[m] = stp[m - 1] * st

    P = {}
    for m in range(0, lmax + 1):
        sgn_df = ((-1.0) ** m) * _dfact(2 * m - 1)
        if m == 0:
            pmm = jnp.ones_like(ct)
        else:
            pmm = sgn_df * stp[m]
        prev, cur = pmm, None
        P[(m, m)] = pmm
        if m < lmax:
            cur = ct * float(2 * m + 1) * pmm
            P[(m + 1, m)] = cur
        for ll in range(m + 2, lmax + 1):
            rcp = 1.0 / float(ll - m)
            nxt = (float(2 * ll - 1) * rcp) * ct * cur - (float(ll + m - 1) * rcp) * prev
            prev, cur = cur, nxt
            P[(ll, m)] = nxt

    comps = []
    for l in _LS:
        for m in range(0, l + 1):
            norm = math.sqrt((2 * l + 1) / (4.0 * math.pi)
                             * math.factorial(l - m) / math.factorial(l + m))
            base = norm * P[(l, m)]
            if m == 0:
                comps.append(base)
            else:
                comps.append(base * er[m])
                comps.append(base * ei[m])
    comps.append(jnp.ones_like(xc))
    return comps


def _node_outputs(q, getw1, getb1, getw2, getb2):
    qls, wls = [], []
    off = 0
    for l, (i1, i2, i3, cs) in zip(_LS, _TABLES):
        qr = [None] * (2 * l + 1)
        qi = [None] * (2 * l + 1)
        zero = jnp.zeros_like(q[off])
        qr[l] = q[off]
        qi[l] = zero
        idx = off + 1
        norm2 = qr[l] * qr[l]
        for m in range(1, l + 1):
            rr, ii = q[idx], q[idx + 1]
            idx += 2
            qr[l + m] = rr
            qi[l + m] = ii
            sgn = (-1.0) ** m
            qr[l - m] = sgn * rr
            qi[l - m] = (-sgn) * ii
            norm2 = norm2 + 2.0 * (rr * rr + ii * ii)
        off = idx
        wsum = zero
        for a, b, c, coef in zip(i1, i2, i3, cs):
            ar, ai = qr[a], qi[a]
            br, bi = qr[b], qi[b]
            cr, ci = qr[c], qi[c]
            tr = ar * br - ai * bi
            ti = ar * bi + ai * br
            wsum = wsum + float(coef) * (tr * cr - ti * ci)
        ql = jnp.sqrt((4.0 * math.pi / (2 * l + 1)) * norm2)
        p = norm2 * jnp.sqrt(norm2)
        wl = jnp.nan_to_num(wsum / p)
        qls.append(ql)
        wls.append(wl)

    emb = qls + wls
    h = []
    for j in range(64):
        pre = getb1(j)
        for k in range(4):
            pre = pre + emb[k] * getw1(k, j)
        h.append(pre * (1.0 / (1.0 + jnp.exp(-pre))))
    outs = []
    for k in range(4):
        o = getb2(k)
        for j in range(64):
            o = o + h[j] * getw2(j, k)
        outs.append(o)
    return outs, emb


def _feat_body(ea_ref, out_ref):
    xc = ea_ref[0, 0]
    yc = ea_ref[0, 1]
    zc = ea_ref[0, 2]
    comps = _edge_comps(xc, yc, zc)
    for j, cmp in enumerate(comps):
        out_ref[j, 0] = cmp
    zero = jnp.zeros_like(xc)
    for j in range(len(comps), _NCOLS):
        out_ref[j, 0] = zero


def _finish_body(a_ref, w1_ref, b1_ref, w2_ref, b2_ref, out_ref, emb_ref):
    count = a_ref[_COUNT_COL]
    inv = 1.0 / jnp.maximum(count, 1.0)
    q = [a_ref[j] * inv for j in range(_NCOMP)]
    outs, emb = _node_outputs(
        q,
        lambda k, j: w1_ref[k, j],
        lambda j: b1_ref[j],
        lambda j, k: w2_ref[j, k],
        lambda k: b2_ref[k],
    )
    for k in range(4):
        out_ref[k] = outs[k]
        emb_ref[k] = emb[k]


def _make_scatter(n_pad, ep, n_chunks):
    mesh = plsc.VectorSubcoreMesh(core_axis_name="c", subcore_axis_name="s",
                                  num_cores=_NC, num_subcores=_NS)

    @functools.partial(
        pl.kernel,
        out_type=jax.ShapeDtypeStruct((_NC * _NS * n_pad,), jnp.float32),
        mesh=mesh,
        scratch_types=[
            pltpu.VMEM((2, _CHUNK), jnp.float32),
            pltpu.VMEM((2, _CHUNK), jnp.int32),
            pltpu.VMEM((n_pad,), jnp.float32),
            pltpu.SemaphoreType.DMA,
            pltpu.SemaphoreType.DMA,
            pltpu.SemaphoreType.DMA,
            pltpu.SemaphoreType.DMA,
        ],
        compiler_params=pltpu.CompilerParams(needs_layout_passes=False),
    )
    def scatter(ft_hbm, dst_hbm, zeros_hbm, out_hbm, vals_v, idx_v, acc,
                vs0, vs1, is0, is1):
        c = lax.axis_index("c")
        s = lax.axis_index("s")
        w = c * _NS + s
        vsem = (vs0, vs1)
        isem = (is0, is1)

        def start(slot, k):
            pltpu.async_copy(ft_hbm.at[pl.ds(w * ep + k * _CHUNK, _CHUNK)],
                             vals_v.at[slot], vsem[slot])
            pltpu.async_copy(dst_hbm.at[pl.ds(k * _CHUNK, _CHUNK)],
                             idx_v.at[slot], isem[slot])

        def wait(slot):
            pltpu.make_async_copy(ft_hbm.at[pl.ds(0, _CHUNK)],
                                  vals_v.at[slot], vsem[slot]).wait()
            pltpu.make_async_copy(dst_hbm.at[pl.ds(0, _CHUNK)],
                                  idx_v.at[slot], isem[slot]).wait()

        def process(slot):
            for i in range(_CHUNK // 16):
                idx = idx_v[slot, pl.ds(i * 16, 16)]
                val = vals_v[slot, pl.ds(i * 16, 16)]
                plsc.addupdate_scatter(acc, [idx], val)

        pltpu.sync_copy(zeros_hbm, acc)
        start(0, 0)

        def chunk_body(k2, carry):
            k0 = 2 * k2
            start(1, k0 + 1)
            wait(0)
            process(0)

            @pl.when(k0 + 2 < n_chunks)
            def _():
                start(0, k0 + 2)

            wait(1)
            process(1)
            return carry

        lax.fori_loop(0, n_chunks // 2, chunk_body, 0)
        pltpu.sync_copy(acc, out_hbm.at[pl.ds(w * n_pad, n_pad)])

    return scatter


def kernel(x, edge_index, edge_attr, W1, b1, W2, b2):
    n = x.shape[0]
    e = edge_attr.shape[0]
    nb = -(-e // 1024)
    cpw = -(-nb // (_NC * _NS))
    nbp = _NC * _NS * cpw
    ep = nbp * 1024
    n_pad = -(-n // 1024) * 1024

    dst = edge_index[1].astype(jnp.int32)
    pad_e = ep - e
    ea_p = jnp.concatenate(
        [edge_attr.astype(jnp.float32),
         jnp.broadcast_to(jnp.array([1.0, 0.0, 0.0], jnp.float32), (pad_e, 3))])
    dst_p = jnp.concatenate([dst, jnp.full((pad_e,), n, jnp.int32)])

    fb = ep // 2048
    ea_t = ea_p.T.reshape(3, fb, 16, 128).transpose(1, 0, 2, 3)
    feat = pl.pallas_call(
        _feat_body,
        grid=(fb,),
        in_specs=[pl.BlockSpec((1, 3, 16, 128), lambda i: (i, 0, 0, 0))],
        out_specs=pl.BlockSpec((_NCOLS, 1, 16, 128), lambda i: (0, i, 0, 0)),
        out_shape=jax.ShapeDtypeStruct((_NCOLS, fb, 16, 128), jnp.float32),
    )(ea_t)
    ft_flat = feat.reshape(_NCOLS * ep)

    n_chunks = ep // _CHUNK
    zeros = jnp.zeros((n_pad,), jnp.float32)
    colsums = _make_scatter(n_pad, ep, n_chunks)(ft_flat, dst_p, zeros)
    a_t = colsums.reshape(_NCOLS, n_pad // 128, 128)

    out_t, emb_t = pl.pallas_call(
        _finish_body,
        grid=(n_pad // 1024,),
        in_specs=[
            pl.BlockSpec((_NCOLS, 8, 128), lambda g: (0, g, 0)),
            pl.BlockSpec(memory_space=pltpu.SMEM),
            pl.BlockSpec(memory_space=pltpu.SMEM),
            pl.BlockSpec(memory_space=pltpu.SMEM),
            pl.BlockSpec(memory_space=pltpu.SMEM),
        ],
        out_specs=[
            pl.BlockSpec((4, 8, 128), lambda g: (0, g, 0)),
            pl.BlockSpec((4, 8, 128), lambda g: (0, g, 0)),
        ],
        out_shape=[
            jax.ShapeDtypeStruct((4, n_pad // 128, 128), jnp.float32),
            jax.ShapeDtypeStruct((4, n_pad // 128, 128), jnp.float32),
        ],
    )(a_t, W1, b1, W2, b2)

    out = out_t.reshape(4, n_pad)[:, :n].T
    emb = emb_t.reshape(4, n_pad)[:, :n].T
    return out, emb


# speedup vs baseline: 12.6479x; 1.0010x over previous
import functools
import math

import numpy as np
import jax
import jax.numpy as jnp
from jax import lax
from jax.experimental import pallas as pl
from jax.experimental.pallas import tpu as pltpu
from jax.experimental.pallas import tpu_sc as plsc

_LS = (4, 6)
_NCOMP = 22
_NCOLS = 32
_COUNT_COL = 22
_NC, _NS = 2, 16
_CHUNK = 4096


def _w3j(j1, j2, j3, m1, m2, m3):
    if m1 + m2 + m3 != 0:
        return 0.0
    f = math.factorial
    delta = math.sqrt(f(j1 + j2 - j3) * f(j1 - j2 + j3) * f(-j1 + j2 + j3)
                      / f(j1 + j2 + j3 + 1))
    pref = delta * math.sqrt(f(j1 + m1) * f(j1 - m1) * f(j2 + m2) * f(j2 - m2)
                             * f(j3 + m3) * f(j3 - m3))
    tmin = max(0, j2 - j3 - m1, j1 - j3 + m2)
    tmax = min(j1 + j2 - j3, j1 - m1, j2 + m2)
    s = 0.0
    for t in range(tmin, tmax + 1):
        s += ((-1.0) ** t) / (f(t) * f(j3 - j2 + t + m1) * f(j3 - j1 + t - m2)
                              * f(j1 + j2 - j3 - t) * f(j1 - t - m1) * f(j2 - t + m2))
    return ((-1.0) ** (j1 - j2 - m3)) * pref * s


def _wigner_table(l):
    i1, i2, i3, c = [], [], [], []
    for m1 in range(-l, l + 1):
        for m2 in range(-l, l + 1):
            m3 = -m1 - m2
            if abs(m3) > l:
                continue
            v = _w3j(l, l, l, m1, m2, m3)
            if v != 0.0:
                i1.append(m1 + l)
                i2.append(m2 + l)
                i3.append(m3 + l)
                c.append(np.float32(v))
    return i1, i2, i3, c


_TABLES = [_wigner_table(l) for l in _LS]


def _dfact(n):
    r = 1.0
    while n > 1:
        r *= n
        n -= 2
    return r


def _edge_comps(xc, yc, zc):
    r2 = xc * xc + yc * yc + zc * zc
    ct = zc * lax.rsqrt(r2)
    st = jnp.sqrt(jnp.clip(1.0 - ct * ct, 0.0, 1.0))
    rho2 = xc * xc + yc * yc
    safe = rho2 > 0.0
    rinv = lax.rsqrt(rho2)
    cp = jnp.where(safe, xc * rinv, 1.0)
    sp = jnp.where(safe, yc * rinv, 0.0)

    lmax = max(_LS)
    er = [None] * (lmax + 1)
    ei = [None] * (lmax + 1)
    er[1], ei[1] = cp, sp
    for m in range(2, lmax + 1):
        er[m] = er[m - 1] * cp - ei[m - 1] * sp
        ei[m] = er[m - 1] * sp + ei[m - 1] * cp

    stp = [None] * (lmax + 1)
    if lmax >= 1:
        stp[1] = st
    for m in range(2, lmax + 1):
        stp[m] = stp[m - 1] * st

    P = {}
    for m in range(0, lmax + 1):
        sgn_df = ((-1.0) ** m) * _dfact(2 * m - 1)
        if m == 0:
            pmm = jnp.ones_like(ct)
        else:
            pmm = sgn_df * stp[m]
        prev, cur = pmm, None
        P[(m, m)] = pmm
        if m < lmax:
            cur = ct * float(2 * m + 1) * pmm
            P[(m + 1, m)] = cur
        for ll in range(m + 2, lmax + 1):
            rcp = 1.0 / float(ll - m)
            nxt = (float(2 * ll - 1) * rcp) * ct * cur - (float(ll + m - 1) * rcp) * prev
            prev, cur = cur, nxt
            P[(ll, m)] = nxt

    comps = []
    for l in _LS:
        for m in range(0, l + 1):
            norm = math.sqrt((2 * l + 1) / (4.0 * math.pi)
                             * math.factorial(l - m) / math.factorial(l + m))
            base = norm * P[(l, m)]
            if m == 0:
                comps.append(base)
            else:
                comps.append(base * er[m])
                comps.append(base * ei[m])
    comps.append(jnp.ones_like(xc))
    return comps


def _node_outputs(q, getw1, getb1, getw2, getb2):
    qls, wls = [], []
    off = 0
    for l, (i1, i2, i3, cs) in zip(_LS, _TABLES):
        qr = [None] * (2 * l + 1)
        qi = [None] * (2 * l + 1)
        zero = jnp.zeros_like(q[off])
        qr[l] = q[off]
        qi[l] = zero
        idx = off + 1
        norm2 = qr[l] * qr[l]
        for m in range(1, l + 1):
            rr, ii = q[idx], q[idx + 1]
            idx += 2
            qr[l + m] = rr
            qi[l + m] = ii
            sgn = (-1.0) ** m
            qr[l - m] = sgn * rr
            qi[l - m] = (-sgn) * ii
            norm2 = norm2 + 2.0 * (rr * rr + ii * ii)
        off = idx
        wsum = zero
        for a, b, c, coef in zip(i1, i2, i3, cs):
            ar, ai = qr[a], qi[a]
            br, bi = qr[b], qi[b]
            cr, ci = qr[c], qi[c]
            tr = ar * br - ai * bi
            ti = ar * bi + ai * br
            wsum = wsum + float(coef) * (tr * cr - ti * ci)
        ql = jnp.sqrt((4.0 * math.pi / (2 * l + 1)) * norm2)
        p = norm2 * jnp.sqrt(norm2)
        wl = jnp.nan_to_num(wsum / p)
        qls.append(ql)
        wls.append(wl)

    emb = qls + wls
    h = []
    for j in range(64):
        pre = getb1(j)
        for k in range(4):
            pre = pre + emb[k] * getw1(k, j)
        h.append(pre * (1.0 / (1.0 + jnp.exp(-pre))))
    outs = []
    for k in range(4):
        o = getb2(k)
        for j in range(64):
            o = o + h[j] * getw2(j, k)
        outs.append(o)
    return outs, emb


def _feat_body(ea_ref, out_ref):
    xc = ea_ref[0, 0]
    yc = ea_ref[0, 1]
    zc = ea_ref[0, 2]
    comps = _edge_comps(xc, yc, zc)
    for j, cmp in enumerate(comps):
        out_ref[j, 0] = cmp
    zero = jnp.zeros_like(xc)
    for j in range(len(comps), _NCOLS):
        out_ref[j, 0] = zero


def _finish_body(a_ref, w1_ref, b1_ref, w2_ref, b2_ref, out_ref, emb_ref):
    count = a_ref[_COUNT_COL]
    inv = 1.0 / jnp.maximum(count, 1.0)
    q = [a_ref[j] * inv for j in range(_NCOMP)]
    outs, emb = _node_outputs(
        q,
        lambda k, j: w1_ref[k, j],
        lambda j: b1_ref[j],
        lambda j, k: w2_ref[j, k],
        lambda k: b2_ref[k],
    )
    for k in range(4):
        out_ref[k] = outs[k]
        emb_ref[k] = emb[k]


def _make_scatter(n_pad, ep, n_chunks):
    mesh = plsc.VectorSubcoreMesh(core_axis_name="c", subcore_axis_name="s",
                                  num_cores=_NC, num_subcores=_NS)

    @functools.partial(
        pl.kernel,
        out_type=jax.ShapeDtypeStruct((_NC * _NS * n_pad,), jnp.float32),
        mesh=mesh,
        scratch_types=[
            pltpu.VMEM((2, _CHUNK), jnp.float32),
            pltpu.VMEM((2, _CHUNK), jnp.int32),
            pltpu.VMEM((n_pad,), jnp.float32),
            pltpu.SemaphoreType.DMA,
            pltpu.SemaphoreType.DMA,
            pltpu.SemaphoreType.DMA,
            pltpu.SemaphoreType.DMA,
        ],
        compiler_params=pltpu.CompilerParams(needs_layout_passes=False),
    )
    def scatter(ft_hbm, dst_hbm, zeros_hbm, out_hbm, vals_v, idx_v, acc,
                vs0, vs1, is0, is1):
        c = lax.axis_index("c")
        s = lax.axis_index("s")
        w = c * _NS + s
        vsem = (vs0, vs1)
        isem = (is0, is1)

        def start(slot, k):
            pltpu.async_copy(ft_hbm.at[pl.ds(w * ep + k * _CHUNK, _CHUNK)],
                             vals_v.at[slot], vsem[slot])
            pltpu.async_copy(dst_hbm.at[pl.ds(k * _CHUNK, _CHUNK)],
                             idx_v.at[slot], isem[slot])

        def wait(slot):
            pltpu.make_async_copy(ft_hbm.at[pl.ds(0, _CHUNK)],
                                  vals_v.at[slot], vsem[slot]).wait()
            pltpu.make_async_copy(dst_hbm.at[pl.ds(0, _CHUNK)],
                                  idx_v.at[slot], isem[slot]).wait()

        def process(slot):
            for i in range(_CHUNK // 16):
                idx = idx_v[slot, pl.ds(i * 16, 16)]
                val = vals_v[slot, pl.ds(i * 16, 16)]
                plsc.addupdate_scatter(acc, [idx], val)

        pltpu.sync_copy(zeros_hbm, acc)
        start(0, 0)

        def chunk_body(k2, carry):
            k0 = 2 * k2
            start(1, k0 + 1)
            wait(0)
            process(0)

            @pl.when(k0 + 2 < n_chunks)
            def _():
                start(0, k0 + 2)

            wait(1)
            process(1)
            return carry

        lax.fori_loop(0, n_chunks // 2, chunk_body, 0)
        pltpu.sync_copy(acc, out_hbm.at[pl.ds(w * n_pad, n_pad)])

    return scatter


def kernel(x, edge_index, edge_attr, W1, b1, W2, b2):
    n = x.shape[0]
    e = edge_attr.shape[0]
    nb = -(-e // 1024)
    cpw = -(-nb // (_NC * _NS))
    nbp = _NC * _NS * cpw
    ep = nbp * 1024
    n_pad = -(-(n + 1) // 1024) * 1024

    dst = edge_index[1].astype(jnp.int32)
    pad_e = ep - e
    ea_p = jnp.concatenate(
        [edge_attr.astype(jnp.float32),
         jnp.broadcast_to(jnp.array([1.0, 0.0, 0.0], jnp.float32), (pad_e, 3))])
    dst_p = jnp.concatenate([dst, jnp.full((pad_e,), n, jnp.int32)])

    fb = ep // 2048
    ea_t = ea_p.T.reshape(3, fb, 16, 128).transpose(1, 0, 2, 3)
    feat = pl.pallas_call(
        _feat_body,
        grid=(fb,),
        in_specs=[pl.BlockSpec((1, 3, 16, 128), lambda i: (i, 0, 0, 0))],
        out_specs=pl.BlockSpec((_NCOLS, 1, 16, 128), lambda i: (0, i, 0, 0)),
        out_shape=jax.ShapeDtypeStruct((_NCOLS, fb, 16, 128), jnp.float32),
    )(ea_t)
    ft_flat = feat.reshape(_NCOLS * ep)

    n_chunks = ep // _CHUNK
    zeros = jnp.zeros((n_pad,), jnp.float32)
    colsums = _make_scatter(n_pad, ep, n_chunks)(ft_flat, dst_p, zeros)
    a_t = colsums.reshape(_NCOLS, n_pad // 128, 128)

    out_t, emb_t = pl.pallas_call(
        _finish_body,
        grid=(n_pad // 1024,),
        in_specs=[
            pl.BlockSpec((_NCOLS, 8, 128), lambda g: (0, g, 0)),
            pl.BlockSpec(memory_space=pltpu.SMEM),
            pl.BlockSpec(memory_space=pltpu.SMEM),
            pl.BlockSpec(memory_space=pltpu.SMEM),
            pl.BlockSpec(memory_space=pltpu.SMEM),
        ],
        out_specs=[
            pl.BlockSpec((4, 8, 128), lambda g: (0, g, 0)),
            pl.BlockSpec((4, 8, 128), lambda g: (0, g, 0)),
        ],
        out_shape=[
            jax.ShapeDtypeStruct((4, n_pad // 128, 128), jnp.float32),
            jax.ShapeDtypeStruct((4, n_pad // 128, 128), jnp.float32),
        ],
    )(a_t, W1, b1, W2, b2)

    out = out_t.reshape(4, n_pad)[:, :n].T
    emb = emb_t.reshape(4, n_pad)[:, :n].T
    return out, emb
